# baseline (speedup 1.0000x reference)
import jax
import jax.numpy as jnp
from jax.experimental import pallas as pl

_EPS = 1e-5


def _bn_relu_cm(y, g, b):
    m = jnp.mean(y, axis=(0, 2), keepdims=True)
    v = jnp.mean((y - m) ** 2, axis=(0, 2), keepdims=True)
    return jnp.maximum(g * (y - m) / jnp.sqrt(v + _EPS) + b, 0.0)


def _enc_body(x_ref, w_ref, b_ref, g_ref, bb_ref, o_ref):
    n, k, hh, ww = x_ref.shape
    x = x_ref[...].reshape(n, k, hh * ww)
    w = w_ref[...]
    y = jnp.stack(
        [jnp.dot(w.T, x[i], preferred_element_type=jnp.float32)
         for i in range(n)], axis=0)
    y = _bn_relu_cm(y + b_ref[...], g_ref[...], bb_ref[...])
    o_ref[...] = y.reshape(n, 16, hh, ww)


def _fold(cm, groups, c):
    s = cm[:, 0:c]
    for i in range(1, groups):
        s = s + cm[:, i * c:(i + 1) * c]
    return jnp.concatenate([s / groups] * groups, axis=1)


def _packed_bn_relu(y, groups, c, g, b):
    cm = _fold(jnp.mean(y, axis=0, keepdims=True), groups, c)
    yc = y - cm
    cv = _fold(jnp.mean(yc * yc, axis=0, keepdims=True), groups, c)
    return jnp.maximum(g * yc / jnp.sqrt(cv + _EPS) + b, 0.0)


def _mid_body(x_ref, w3_ref, b3_ref, g3_ref, bb3_ref,
              w4_ref, b4_ref, g4_ref, bb4_ref,
              embt_ref, emb_ref,
              wd_ref, bd_ref, gd_ref, bbd_ref,
              ze_ref, lat_ref, zq_ref, hd_ref):
    h3 = jnp.dot(x_ref[...], w3_ref[...], preferred_element_type=jnp.float32)
    h3 = _packed_bn_relu(h3 + b3_ref[...], 1, 16, g3_ref[...], bb3_ref[...])
    z = jnp.dot(h3, w4_ref[...], preferred_element_type=jnp.float32)
    z = _packed_bn_relu(z + b4_ref[...], 1, 32, g4_ref[...], bb4_ref[...])
    ze_ref[...] = z

    m = z.shape[0]
    k = emb_ref.shape[0]
    embt = embt_ref[...]
    acc = jnp.zeros((m, k), jnp.float32)
    for c in range(32):
        d = z[:, c:c + 1] - embt[c:c + 1, :]
        acc = acc + d * d
    dmin = jnp.min(acc, axis=1, keepdims=True)
    iota = jax.lax.broadcasted_iota(jnp.int32, (m, k), 1)
    lat = jnp.min(jnp.where(acc == dmin, iota, k), axis=1, keepdims=True)
    lat_ref[...] = lat

    onehot = (iota == lat).astype(jnp.float32)
    zq = jnp.dot(onehot, emb_ref[...], preferred_element_type=jnp.float32)
    zq_ref[...] = zq
    hd = jnp.dot(zq, wd_ref[...], preferred_element_type=jnp.float32)
    hd_ref[...] = _packed_bn_relu(hd + bd_ref[...], 1, 16,
                                  gd_ref[...], bbd_ref[...])


def _deccls_body(x_ref, w_ref, b_ref, g_ref, bb_ref, o_ref):
    n, kc, hh, ww = x_ref.shape
    k = kc // 4
    x = x_ref[...].reshape(n, kc, hh * ww)
    w = w_ref[...]
    ys = []
    for cls in range(4):
        wc = w[cls]
        ys.append(jnp.stack(
            [jnp.dot(wc.T, x[i, cls * k:(cls + 1) * k, :],
                     preferred_element_type=jnp.float32)
             for i in range(n)], axis=0))
    y = jnp.stack(ys, axis=0) + b_ref[...]
    m = jnp.mean(y, axis=(0, 1, 3), keepdims=True)
    v = jnp.mean((y - m) ** 2, axis=(0, 1, 3), keepdims=True)
    o_ref[...] = jnp.maximum(
        g_ref[...] * (y - m) / jnp.sqrt(v + _EPS) + bb_ref[...], 0.0)


def _dec4_body(x_ref, w4_ref, b4_ref, g4_ref, bb4_ref, wo_ref, bo_ref,
               o_ref):
    n = 4
    y3 = x_ref[...]
    w4 = w4_ref[...]
    ys = []
    for c3 in range(4):
        yc = y3[c3]
        ys.append(jnp.stack(
            [jnp.dot(w4.T, yc[i], preferred_element_type=jnp.float32)
             for i in range(n)], axis=0))
    y = jnp.stack(ys, axis=0) + b4_ref[...]
    cm = jnp.mean(y, axis=(0, 1, 3), keepdims=True)
    m16 = (cm[:, :, 0:16] + cm[:, :, 16:32]
           + cm[:, :, 32:48] + cm[:, :, 48:64]) * 0.25
    mt = jnp.concatenate([m16] * 4, axis=2)
    yc = y - mt
    cv = jnp.mean(yc * yc, axis=(0, 1, 3), keepdims=True)
    v16 = (cv[:, :, 0:16] + cv[:, :, 16:32]
           + cv[:, :, 32:48] + cv[:, :, 48:64]) * 0.25
    vt = jnp.concatenate([v16] * 4, axis=2)
    yr = jnp.maximum(
        g4_ref[...] * yc / jnp.sqrt(vt + _EPS) + bb4_ref[...], 0.0)
    wo = wo_ref[...]
    outs = []
    for c3 in range(4):
        yrc = yr[c3]
        outs.append(jnp.stack(
            [jnp.dot(wo.T, yrc[i], preferred_element_type=jnp.float32)
             for i in range(n)], axis=0))
    o_ref[...] = jnp.stack(outs, axis=0) + bo_ref[...]


def _asm_body(p_ref, o_ref):
    p = p_ref[...]
    for co in range(3):
        ms = []
        for r in range(4):
            a3, a4 = r // 2, r % 2
            cols = [p[:, 2 * a3 + b3, 0, 3 * (2 * a4 + b4) + co, :]
                    for b3 in range(2) for b4 in range(2)]
            ms.append(jnp.stack(cols, axis=-1).reshape(57, 228))
        o_ref[0, co] = jnp.stack(ms, axis=1).reshape(228, 228)


def _taps_s2(xp, k, ho):
    cols = []
    for kh in range(k):
        for kw in range(k):
            cols.append(xp[:, :, kh:kh + 2 * ho - 1:2, kw:kw + 2 * ho - 1:2])
    return jnp.concatenate(cols, axis=1)


def _wmat_cm(w):
    co, ci, kh, kw = w.shape
    return jnp.transpose(w, (2, 3, 1, 0)).reshape(kh * kw * ci, co)


def _c3(a):
    return a.reshape(1, -1, 1)


def kernel(x, params):
    p = params
    n = x.shape[0]

    xp = jnp.pad(x, ((0, 0), (0, 0), (1, 1), (1, 1)))
    x1 = _taps_s2(xp, 4, 112)
    h1 = pl.pallas_call(
        _enc_body,
        out_shape=jax.ShapeDtypeStruct((n, 16, 112, 112), jnp.float32),
    )(x1, _wmat_cm(p['ew1']), _c3(p['eb1']), _c3(p['eg1']), _c3(p['ebb1']))

    h1p = jnp.pad(h1, ((0, 0), (0, 0), (1, 1), (1, 1)))
    x2 = _taps_s2(h1p, 4, 56)
    h2 = pl.pallas_call(
        _enc_body,
        out_shape=jax.ShapeDtypeStruct((n, 16, 56, 56), jnp.float32),
    )(x2, _wmat_cm(p['ew2']), _c3(p['eb2']), _c3(p['eg2']), _c3(p['ebb2']))

    h2n = jnp.transpose(h2, (0, 2, 3, 1))
    h2np = jnp.pad(h2n, ((0, 0), (1, 1), (1, 1), (0, 0)))
    cols = [h2np[:, kh:kh + 57:2, kw:kw + 57:2, :]
            for kh in range(2) for kw in range(2)]
    x3 = jnp.concatenate(cols, axis=-1).reshape(n * 841, 64)

    def r2(a):
        return a.reshape(1, -1)

    w4 = jnp.transpose(p['ew4'][:, :, 0, 0], (1, 0))
    wd1 = p['dw1'][:, :, 0, 0]
    m3 = n * 841
    ze_f, lat_f, zq_f, hd_f = pl.pallas_call(
        _mid_body,
        out_shape=(jax.ShapeDtypeStruct((m3, 32), jnp.float32),
                   jax.ShapeDtypeStruct((m3, 1), jnp.int32),
                   jax.ShapeDtypeStruct((m3, 32), jnp.float32),
                   jax.ShapeDtypeStruct((m3, 16), jnp.float32)),
    )(x3, _wmat_cm(p['ew3']), r2(p['eb3']), r2(p['eg3']), r2(p['ebb3']),
      w4, r2(p['eb4']), r2(p['eg4']), r2(p['ebb4']),
      jnp.transpose(p['emb'], (1, 0)), p['emb'],
      wd1, r2(p['db1']), r2(p['dg1']), r2(p['dbb1']))

    ze = jnp.transpose(ze_f.reshape(n, 29, 29, 32), (0, 3, 1, 2))
    zq = jnp.transpose(zq_f.reshape(n, 29, 29, 32), (0, 3, 1, 2))
    latents = lat_f.reshape(n, 29, 29)
    hd1 = jnp.transpose(hd_f.reshape(n, 29, 29, 16), (0, 3, 1, 2))
    hdp = jnp.pad(hd1, ((0, 0), (0, 0), (1, 1), (1, 1)))

    w2t = p['dw2']
    w2c = jnp.stack([
        jnp.concatenate(
            [w2t[:, :, 3 - a - 2 * rh, 3 - b - 2 * rw]
             for rh in range(2) for rw in range(2)], axis=0)
        for a in range(2) for b in range(2)], axis=0)
    w3c = jnp.stack([p['dw3'][:, :, 1 - a, 1 - b]
                     for a in range(2) for b in range(2)], axis=0)
    w4cat = jnp.concatenate(
        [p['dw4'][:, :, a, b] for a in range(2) for b in range(2)],
        axis=1)
    wo = p['ow'][:, :, 0, 0]
    zo = jnp.zeros_like(wo)
    wo_blk = jnp.concatenate([
        jnp.concatenate([wo if i == j else zo for j in range(4)], axis=1)
        for i in range(4)], axis=0)

    def c4(a, reps=1):
        a = jnp.tile(a, reps) if reps > 1 else a
        return a.reshape(1, 1, -1, 1)

    x2d = jnp.concatenate(
        [hdp[:, :, a + rh:a + rh + 29, b + rw:b + rw + 29]
         for a in range(2) for b in range(2)
         for rh in range(2) for rw in range(2)], axis=1)
    y2 = pl.pallas_call(
        _deccls_body,
        out_shape=jax.ShapeDtypeStruct((4, n, 16, 841), jnp.float32),
    )(x2d, w2c, c4(p['db2']), c4(p['dg2']), c4(p['dbb2']))
    d2 = jnp.transpose(
        y2.reshape(2, 2, n, 16, 29, 29),
        (2, 3, 4, 0, 5, 1)).reshape(n, 16, 58, 58)

    x3d = jnp.concatenate(
        [d2[:, :, a:a + 57, b:b + 57]
         for a in range(2) for b in range(2)], axis=1)
    y3 = pl.pallas_call(
        _deccls_body,
        out_shape=jax.ShapeDtypeStruct((4, n, 16, 3249), jnp.float32),
    )(x3d, w3c, c4(p['db3']), c4(p['dg3']), c4(p['dbb3']))

    ot = pl.pallas_call(
        _dec4_body,
        out_shape=jax.ShapeDtypeStruct((4, n, 12, 3249), jnp.float32),
    )(y3, w4cat, c4(p['db4'], 4), c4(p['dg4'], 4), c4(p['dbb4'], 4),
      wo_blk, c4(p['ob'], 4))

    ot5 = jnp.transpose(ot.reshape(4, n, 12, 57, 57), (3, 0, 1, 2, 4))
    x_tilde = pl.pallas_call(
        _asm_body,
        grid=(n,),
        in_specs=[pl.BlockSpec((57, 4, 1, 12, 57),
                               lambda i: (0, 0, i, 0, 0))],
        out_specs=pl.BlockSpec((1, 3, 228, 228), lambda i: (i, 0, 0, 0)),
        out_shape=jax.ShapeDtypeStruct((n, 3, 228, 228), jnp.float32),
    )(ot5)

    return x_tilde, ze, zq, latents
